# Initial kernel scaffold; baseline (speedup 1.0000x reference)
import jax
import jax.numpy as jnp
from jax.experimental import pallas as pl


def kernel(item_ids, user_ids, item_table, channel_tables, user_table):
    raise NotImplementedError("write your pallas kernel here")


# speedup vs baseline: 3.2280x; 3.2280x over previous
import functools

import jax
import jax.numpy as jnp
from jax import lax
from jax.experimental import pallas as pl
from jax.experimental.pallas import tpu as pltpu
from jax.experimental.pallas import tpu_sc as plsc

EMBED_DIM = 32
NUM_CORES = 2
NUM_SUBCORES = 16
NUM_WORKERS = NUM_CORES * NUM_SUBCORES
LANES = 16


def _make_sc_lookup(F: int, chunk: int):
    assert F % NUM_WORKERS == 0
    per_w = F // NUM_WORKERS
    assert per_w % chunk == 0 and chunk % 8 == 0
    n_chunks = per_w // chunk
    mesh = plsc.VectorSubcoreMesh(core_axis_name="c", subcore_axis_name="s")

    @functools.partial(
        pl.kernel,
        mesh=mesh,
        out_type=jax.ShapeDtypeStruct((F, EMBED_DIM), jnp.float32),
        compiler_params=pltpu.CompilerParams(use_tc_tiling_on_sc=False),
        scratch_types=[
            pltpu.VMEM((chunk,), jnp.int32),
            pltpu.VMEM((chunk,), jnp.int32),
            pltpu.VMEM((chunk, EMBED_DIM), jnp.float32),
            pltpu.VMEM((chunk, EMBED_DIM), jnp.float32),
            pltpu.VMEM((chunk, EMBED_DIM), jnp.float32),
            pltpu.VMEM((chunk, EMBED_DIM), jnp.float32),
            pltpu.VMEM((chunk, EMBED_DIM), jnp.float32),
            pltpu.VMEM((chunk, EMBED_DIM), jnp.float32),
            pltpu.SemaphoreType.DMA,
        ],
    )
    def sc_lookup(ids_hbm, uidx_hbm, item_hbm, ch0_hbm, ch1_hbm, ch2_hbm,
                  user_hbm, out_hbm, idx_v, uidx_v, it_v, c0_v, c1_v, c2_v,
                  us_v, out_v, sem):
        wid = lax.axis_index("s") * NUM_CORES + lax.axis_index("c")
        base = wid * per_w

        def do_chunk(k, _):
            row0 = pl.multiple_of(base + k * chunk, 8)
            pltpu.sync_copy(ids_hbm.at[pl.ds(row0, chunk)], idx_v)
            pltpu.sync_copy(uidx_hbm.at[pl.ds(row0, chunk)], uidx_v)
            cps = [
                pltpu.async_copy(item_hbm.at[idx_v], it_v, sem),
                pltpu.async_copy(ch0_hbm.at[idx_v], c0_v, sem),
                pltpu.async_copy(ch1_hbm.at[idx_v], c1_v, sem),
                pltpu.async_copy(ch2_hbm.at[idx_v], c2_v, sem),
                pltpu.async_copy(user_hbm.at[uidx_v], us_v, sem),
            ]
            for cp in cps:
                cp.wait()

            def row_sum(r, _):
                for h in (0, LANES):
                    s = pl.ds(h, LANES)
                    out_v[r, s] = (it_v[r, s] + c0_v[r, s] + c1_v[r, s]
                                   + c2_v[r, s] + us_v[r, s])
                return 0

            lax.fori_loop(0, chunk, row_sum, 0, unroll=4)
            pltpu.sync_copy(out_v, out_hbm.at[pl.ds(row0, chunk)])
            return 0

        lax.fori_loop(0, n_chunks, do_chunk, 0)

    return sc_lookup


def kernel(item_ids, user_ids, item_table, channel_tables, user_table):
    B, L = item_ids.shape
    F = B * L
    flat_ids = item_ids.reshape(F)
    uidx = jnp.repeat(user_ids, L)
    ch0 = channel_tables[0]
    ch1 = channel_tables[1]
    ch2 = channel_tables[2]
    lookup = _make_sc_lookup(F, chunk=400)
    out = lookup(flat_ids, uidx, item_table, ch0, ch1, ch2, user_table)
    return out.reshape(B, L, EMBED_DIM)


# speedup vs baseline: 10.3262x; 3.1989x over previous
import functools

import jax
import jax.numpy as jnp
from jax import lax
from jax.experimental import pallas as pl
from jax.experimental.pallas import tpu as pltpu
from jax.experimental.pallas import tpu_sc as plsc

EMBED_DIM = 32
NUM_CORES = 2
NUM_SUBCORES = 16
NUM_WORKERS = NUM_CORES * NUM_SUBCORES
LANES = 16
BLK = 2048
SUB = 512


def _transpose_pack(x):
    return jnp.concatenate(
        [x[:, a * SUB:(a + 1) * SUB].T for a in range(4)], axis=1)


def _fuse4_block(a_ref, ch_ref, out_ref):
    out_ref[...] = _transpose_pack(
        a_ref[...] + ch_ref[0] + ch_ref[1] + ch_ref[2])


def _compact1_block(a_ref, out_ref):
    out_ref[...] = _transpose_pack(a_ref[...])


def _compact_tables(tc_body, in_specs, n_blocks):
    out_spec = pl.BlockSpec((SUB, 4 * EMBED_DIM), lambda i: (i, 0))
    return pl.pallas_call(
        tc_body,
        grid=(n_blocks,),
        in_specs=in_specs,
        out_specs=out_spec,
        out_shape=jax.ShapeDtypeStruct((n_blocks * SUB, 4 * EMBED_DIM),
                                       jnp.float32),
    )


def _perm(idx):
    blk = idx // BLK
    rem = idx - blk * BLK
    return blk * BLK + (rem % SUB) * 4 + rem // SUB


def _make_sc_lookup(F: int, chunk: int):
    assert F % NUM_WORKERS == 0
    per_w = F // NUM_WORKERS
    assert per_w % chunk == 0 and chunk % 8 == 0
    n_chunks = per_w // chunk
    mesh = plsc.VectorSubcoreMesh(core_axis_name="c", subcore_axis_name="s")

    @functools.partial(
        pl.kernel,
        mesh=mesh,
        out_type=jax.ShapeDtypeStruct((F, EMBED_DIM), jnp.float32),
        compiler_params=pltpu.CompilerParams(use_tc_tiling_on_sc=False),
        scratch_types=[
            pltpu.VMEM((chunk,), jnp.int32),
            pltpu.VMEM((chunk,), jnp.int32),
            pltpu.VMEM((chunk, EMBED_DIM), jnp.float32),
            pltpu.VMEM((chunk, EMBED_DIM), jnp.float32),
            pltpu.VMEM((chunk, EMBED_DIM), jnp.float32),
            pltpu.SemaphoreType.DMA,
        ],
    )
    def sc_lookup(ids_hbm, uidx_hbm, g_hbm, u_hbm, out_hbm,
                  idx_v, uidx_v, g_v, u_v, out_v, sem):
        wid = lax.axis_index("s") * NUM_CORES + lax.axis_index("c")
        base = wid * per_w

        def do_chunk(k, _):
            row0 = pl.multiple_of(base + k * chunk, 8)
            pltpu.sync_copy(ids_hbm.at[pl.ds(row0, chunk)], idx_v)
            pltpu.sync_copy(uidx_hbm.at[pl.ds(row0, chunk)], uidx_v)
            cps = [
                pltpu.async_copy(g_hbm.at[idx_v], g_v, sem),
                pltpu.async_copy(u_hbm.at[uidx_v], u_v, sem),
            ]
            for cp in cps:
                cp.wait()

            def row_sum(r, _):
                for h in (0, LANES):
                    s = pl.ds(h, LANES)
                    out_v[r, s] = g_v[r, s] + u_v[r, s]
                return 0

            lax.fori_loop(0, chunk, row_sum, 0, unroll=8)
            pltpu.sync_copy(out_v, out_hbm.at[pl.ds(row0, chunk)])
            return 0

        lax.fori_loop(0, n_chunks, do_chunk, 0)

    return sc_lookup


def kernel(item_ids, user_ids, item_table, channel_tables, user_table):
    B, L = item_ids.shape
    F = B * L
    V = item_table.shape[0]
    nb_g = pl.cdiv(V, BLK)
    VU_pad = 100352
    nb_u = VU_pad // BLK

    it_t = item_table.T
    ch_t = jnp.swapaxes(channel_tables, 1, 2)
    us_t = jnp.pad(user_table.T, ((0, 0), (0, VU_pad - user_table.shape[0])))

    it_spec = pl.BlockSpec((EMBED_DIM, BLK), lambda i: (0, i))
    ch_spec = pl.BlockSpec((3, EMBED_DIM, BLK), lambda i: (0, 0, i))
    g_c = _compact_tables(_fuse4_block, [it_spec, ch_spec], nb_g)(it_t, ch_t)
    u_c = _compact_tables(_compact1_block, [it_spec], nb_u)(us_t)

    flat_ids = _perm(item_ids.reshape(F))
    uidx = _perm(jnp.repeat(user_ids, L))

    lookup = _make_sc_lookup(F, chunk=800)
    out = lookup(flat_ids, uidx,
                 g_c.reshape(nb_g * BLK, EMBED_DIM),
                 u_c.reshape(VU_pad, EMBED_DIM))
    return out.reshape(B, L, EMBED_DIM)


# speedup vs baseline: 13.2272x; 1.2809x over previous
import functools

import jax
import jax.numpy as jnp
from jax import lax
from jax.experimental import pallas as pl
from jax.experimental.pallas import tpu as pltpu
from jax.experimental.pallas import tpu_sc as plsc

EMBED_DIM = 32
NUM_CORES = 2
NUM_SUBCORES = 16
NUM_WORKERS = NUM_CORES * NUM_SUBCORES
LANES = 16
BLK = 8192
SUB = 2048


def _store_transposed(x, out_ref):
    eye = jnp.eye(EMBED_DIM, dtype=jnp.float32)
    y = lax.dot_general(x, eye, (((0,), (0,)), ((), ())),
                        preferred_element_type=jnp.float32)
    for a in range(4):
        out_ref[:, a * EMBED_DIM:(a + 1) * EMBED_DIM] = (
            y[a * SUB:(a + 1) * SUB, :])


def _fuse4_block(a_ref, ch_ref, out_ref):
    _store_transposed(a_ref[...] + ch_ref[0] + ch_ref[1] + ch_ref[2], out_ref)


def _compact1_block(a_ref, out_ref):
    _store_transposed(a_ref[...], out_ref)


def _compact_tables(tc_body, in_specs, n_blocks):
    out_spec = pl.BlockSpec((SUB, 4 * EMBED_DIM), lambda i: (i, 0))
    return pl.pallas_call(
        tc_body,
        grid=(n_blocks,),
        in_specs=in_specs,
        out_specs=out_spec,
        out_shape=jax.ShapeDtypeStruct((n_blocks * SUB, 4 * EMBED_DIM),
                                       jnp.float32),
    )


def _perm(idx):
    blk = idx // BLK
    rem = idx - blk * BLK
    return blk * BLK + (rem % SUB) * 4 + rem // SUB


def _make_sc_lookup(F: int, chunk: int):
    assert F % NUM_WORKERS == 0
    per_w = F // NUM_WORKERS
    assert per_w % chunk == 0 and chunk % 8 == 0
    n_chunks = per_w // chunk
    mesh = plsc.VectorSubcoreMesh(core_axis_name="c", subcore_axis_name="s")

    @functools.partial(
        pl.kernel,
        mesh=mesh,
        out_type=jax.ShapeDtypeStruct((F, EMBED_DIM), jnp.float32),
        compiler_params=pltpu.CompilerParams(use_tc_tiling_on_sc=False),
        scratch_types=[
            pltpu.VMEM((chunk,), jnp.int32),
            pltpu.VMEM((chunk,), jnp.int32),
            pltpu.VMEM((chunk, EMBED_DIM), jnp.float32),
            pltpu.VMEM((chunk, EMBED_DIM), jnp.float32),
            pltpu.VMEM((chunk, EMBED_DIM), jnp.float32),
            pltpu.SemaphoreType.DMA,
        ],
    )
    def sc_lookup(ids_hbm, uidx_hbm, g_hbm, u_hbm, out_hbm,
                  idx_v, uidx_v, g_v, u_v, out_v, sem):
        wid = lax.axis_index("s") * NUM_CORES + lax.axis_index("c")
        base = wid * per_w

        def do_chunk(k, _):
            row0 = pl.multiple_of(base + k * chunk, 8)
            pltpu.sync_copy(ids_hbm.at[pl.ds(row0, chunk)], idx_v)
            pltpu.sync_copy(uidx_hbm.at[pl.ds(row0, chunk)], uidx_v)
            cps = [
                pltpu.async_copy(g_hbm.at[idx_v], g_v, sem),
                pltpu.async_copy(u_hbm.at[uidx_v], u_v, sem),
            ]
            for cp in cps:
                cp.wait()

            def row_sum(r, _):
                for h in (0, LANES):
                    s = pl.ds(h, LANES)
                    out_v[r, s] = g_v[r, s] + u_v[r, s]
                return 0

            lax.fori_loop(0, chunk, row_sum, 0, unroll=8)
            pltpu.sync_copy(out_v, out_hbm.at[pl.ds(row0, chunk)])
            return 0

        lax.fori_loop(0, n_chunks, do_chunk, 0)

    return sc_lookup


def kernel(item_ids, user_ids, item_table, channel_tables, user_table):
    B, L = item_ids.shape
    F = B * L
    V = item_table.shape[0]
    nb_g = pl.cdiv(V, BLK)
    VU_pad = 106496
    nb_u = VU_pad // BLK

    it_t = item_table.T
    ch_t = jnp.swapaxes(channel_tables, 1, 2)
    us_t = jnp.pad(user_table.T, ((0, 0), (0, VU_pad - user_table.shape[0])))

    it_spec = pl.BlockSpec((EMBED_DIM, BLK), lambda i: (0, i))
    ch_spec = pl.BlockSpec((3, EMBED_DIM, BLK), lambda i: (0, 0, i))
    g_c = _compact_tables(_fuse4_block, [it_spec, ch_spec], nb_g)(it_t, ch_t)
    u_c = _compact_tables(_compact1_block, [it_spec], nb_u)(us_t)

    flat_ids = _perm(item_ids.reshape(F))
    uidx = _perm(jnp.repeat(user_ids, L))

    lookup = _make_sc_lookup(F, chunk=800)
    out = lookup(flat_ids, uidx,
                 g_c.reshape(nb_g * BLK, EMBED_DIM),
                 u_c.reshape(VU_pad, EMBED_DIM))
    return out.reshape(B, L, EMBED_DIM)


# speedup vs baseline: 20.4068x; 1.5428x over previous
import functools

import jax
import jax.numpy as jnp
from jax import lax
from jax.experimental import pallas as pl
from jax.experimental.pallas import tpu as pltpu
from jax.experimental.pallas import tpu_sc as plsc

EMBED_DIM = 32
NUM_CORES = 2
NUM_SUBCORES = 16
NUM_WORKERS = NUM_CORES * NUM_SUBCORES
LANES = 16
BLK = 8192
SUB = 2048
B = 4096
L = 50
F = B * L


def _transpose_mxu(x):
    eye = jnp.eye(EMBED_DIM, dtype=jnp.float32)
    return lax.dot_general(x, eye, (((0,), (0,)), ((), ())),
                           preferred_element_type=jnp.float32)


def _store_transposed(x, out_ref):
    y = _transpose_mxu(x)
    for a in range(4):
        out_ref[:, a * EMBED_DIM:(a + 1) * EMBED_DIM] = (
            y[a * SUB:(a + 1) * SUB, :])


def _fuse4_block(a_ref, ch_ref, out_ref):
    _store_transposed(a_ref[...] + ch_ref[0] + ch_ref[1] + ch_ref[2], out_ref)


def _compact1_block(a_ref, out_ref):
    _store_transposed(a_ref[...], out_ref)


def _compact_tables(tc_body, in_specs, n_blocks):
    out_spec = pl.BlockSpec((SUB, 4 * EMBED_DIM), lambda i: (i, 0))
    return pl.pallas_call(
        tc_body,
        grid=(n_blocks,),
        in_specs=in_specs,
        out_specs=out_spec,
        out_shape=jax.ShapeDtypeStruct((n_blocks * SUB, 4 * EMBED_DIM),
                                       jnp.float32),
    )


def _perm(idx):
    blk = idx // BLK
    rem = idx - blk * BLK
    return blk * BLK + (rem % SUB) * 4 + rem // SUB


def _epilogue_block(x_ref, out_ref):
    eye = jnp.eye(EMBED_DIM, dtype=jnp.float32)
    for a in range(4):
        xa = x_ref[:, a * EMBED_DIM:(a + 1) * EMBED_DIM]
        out_ref[0, :, a * 1024:(a + 1) * 1024] = lax.dot_general(
            eye, xa, (((1,), (1,)), ((), ())),
            preferred_element_type=jnp.float32)


_epilogue = pl.pallas_call(
    _epilogue_block,
    grid=(L,),
    in_specs=[pl.BlockSpec((1024, 128), lambda i: (i, 0))],
    out_specs=pl.BlockSpec((1, EMBED_DIM, B), lambda i: (i, 0, 0)),
    out_shape=jax.ShapeDtypeStruct((L, EMBED_DIM, B), jnp.float32),
)


def _make_sc_lookup(chunk: int):
    per_w = F // NUM_WORKERS
    assert per_w % chunk == 0 and chunk % 8 == 0
    n_chunks = per_w // chunk
    out_rows = chunk // 4
    mesh = plsc.VectorSubcoreMesh(core_axis_name="c", subcore_axis_name="s")

    @functools.partial(
        pl.kernel,
        mesh=mesh,
        out_type=jax.ShapeDtypeStruct((F // 4, 4 * EMBED_DIM), jnp.float32),
        compiler_params=pltpu.CompilerParams(use_tc_tiling_on_sc=False),
        scratch_types=(
            [pltpu.VMEM((chunk,), jnp.int32)] * 2
            + [pltpu.VMEM((chunk,), jnp.int32)] * 2
            + [pltpu.VMEM((chunk, EMBED_DIM), jnp.float32)] * 2
            + [pltpu.VMEM((chunk, EMBED_DIM), jnp.float32)] * 2
            + [pltpu.VMEM((out_rows, 4 * EMBED_DIM), jnp.float32)] * 2
            + [pltpu.SemaphoreType.DMA] * 3
        ),
    )
    def sc_lookup(ids_hbm, uidx_hbm, g_hbm, u_hbm, out_hbm,
                  idx0, idx1, uidx0, uidx1, g0, g1, u0, u1, o0, o1,
                  sem_i, sem_g, sem_o):
        idx_v = (idx0, idx1)
        uidx_v = (uidx0, uidx1)
        g_v = (g0, g1)
        u_v = (u0, u1)
        out_v = (o0, o1)
        wid = lax.axis_index("s") * NUM_CORES + lax.axis_index("c")
        base = wid * per_w

        def fire_idx(k):
            row0 = pl.multiple_of(base + k * chunk, 8)
            p = k % 2
            return (pltpu.async_copy(ids_hbm.at[pl.ds(row0, chunk)],
                                     idx_v[p], sem_i),
                    pltpu.async_copy(uidx_hbm.at[pl.ds(row0, chunk)],
                                     uidx_v[p], sem_i))

        def fire_gather(k):
            p = k % 2
            return (pltpu.async_copy(g_hbm.at[idx_v[p]], g_v[p], sem_g),
                    pltpu.async_copy(u_hbm.at[uidx_v[p]], u_v[p], sem_g))

        pend_i = fire_idx(0)
        for cp in pend_i:
            cp.wait()
        cur_g = fire_gather(0)
        pend_i = fire_idx(1) if n_chunks > 1 else None
        pend_o = [None, None]
        for k in range(n_chunks):
            p = k % 2
            next_g = None
            if k + 1 < n_chunks:
                for cp in pend_i:
                    cp.wait()
                next_g = fire_gather(k + 1)
            for cp in cur_g:
                cp.wait()
            if k + 2 < n_chunks:
                pend_i = fire_idx(k + 2)
            if pend_o[p] is not None:
                pend_o[p].wait()

            def row_sum(rr, _):
                for a in range(4):
                    for h in (0, LANES):
                        s = pl.ds(h, LANES)
                        d = pl.ds(a * EMBED_DIM + h, LANES)
                        out_v[p][rr, d] = (g_v[p][4 * rr + a, s]
                                           + u_v[p][4 * rr + a, s])
                return 0

            lax.fori_loop(0, out_rows, row_sum, 0, unroll=4)
            orow0 = pl.multiple_of((base + k * chunk) // 4, 8)
            pend_o[p] = pltpu.async_copy(
                out_v[p], out_hbm.at[pl.ds(orow0, out_rows)], sem_o)
            cur_g = next_g
        for po in pend_o:
            if po is not None:
                po.wait()

    return sc_lookup


def kernel(item_ids, user_ids, item_table, channel_tables, user_table):
    V = item_table.shape[0]
    nb_g = pl.cdiv(V, BLK)
    VU_pad = 106496
    nb_u = VU_pad // BLK

    it_t = item_table.T
    ch_t = jnp.swapaxes(channel_tables, 1, 2)
    us_t = jnp.pad(user_table.T, ((0, 0), (0, VU_pad - user_table.shape[0])))

    it_spec = pl.BlockSpec((EMBED_DIM, BLK), lambda i: (0, i))
    ch_spec = pl.BlockSpec((3, EMBED_DIM, BLK), lambda i: (0, 0, i))
    g_c = _compact_tables(_fuse4_block, [it_spec, ch_spec], nb_g)(it_t, ch_t)
    u_c = _compact_tables(_compact1_block, [it_spec], nb_u)(us_t)

    v = jnp.arange(B, dtype=jnp.int32)
    b_of_v = (v % 4) * (B // 4) + v // 4
    flat_ids = _perm(item_ids[b_of_v, :].T.reshape(F))
    uidx = _perm(jnp.broadcast_to(user_ids[b_of_v], (L, B)).reshape(F))

    lookup = _make_sc_lookup(chunk=640)
    packed = lookup(flat_ids, uidx,
                    g_c.reshape(nb_g * BLK, EMBED_DIM),
                    u_c.reshape(VU_pad, EMBED_DIM))
    out_phys = _epilogue(packed)
    return jnp.transpose(out_phys, (2, 0, 1))


# speedup vs baseline: 20.6427x; 1.0116x over previous
import functools

import jax
import jax.numpy as jnp
from jax import lax
from jax.experimental import pallas as pl
from jax.experimental.pallas import tpu as pltpu
from jax.experimental.pallas import tpu_sc as plsc

EMBED_DIM = 32
NUM_CORES = 2
NUM_SUBCORES = 16
NUM_WORKERS = NUM_CORES * NUM_SUBCORES
LANES = 16
BLK = 16384
SUB = 4096
B = 4096
L = 50
F = B * L


def _transpose_mxu(x):
    eye = jnp.eye(EMBED_DIM, dtype=jnp.float32)
    return lax.dot_general(x, eye, (((0,), (0,)), ((), ())),
                           preferred_element_type=jnp.float32)


def _store_transposed(x, out_ref):
    y = _transpose_mxu(x)
    for a in range(4):
        out_ref[:, a * EMBED_DIM:(a + 1) * EMBED_DIM] = (
            y[a * SUB:(a + 1) * SUB, :])


def _fuse4_block(a_ref, ch_ref, out_ref):
    _store_transposed(a_ref[...] + ch_ref[0] + ch_ref[1] + ch_ref[2], out_ref)


def _compact1_block(a_ref, out_ref):
    _store_transposed(a_ref[...], out_ref)


def _compact_tables(tc_body, in_specs, n_blocks):
    out_spec = pl.BlockSpec((SUB, 4 * EMBED_DIM), lambda i: (i, 0))
    return pl.pallas_call(
        tc_body,
        grid=(n_blocks,),
        in_specs=in_specs,
        out_specs=out_spec,
        out_shape=jax.ShapeDtypeStruct((n_blocks * SUB, 4 * EMBED_DIM),
                                       jnp.float32),
    )


def _perm(idx):
    blk = idx // BLK
    rem = idx - blk * BLK
    return blk * BLK + (rem % SUB) * 4 + rem // SUB


def _epilogue_block(x_ref, out_ref):
    eye = jnp.eye(EMBED_DIM, dtype=jnp.float32)
    for a in range(4):
        xa = x_ref[:, a * EMBED_DIM:(a + 1) * EMBED_DIM]
        out_ref[0, :, a * 1024:(a + 1) * 1024] = lax.dot_general(
            eye, xa, (((1,), (1,)), ((), ())),
            preferred_element_type=jnp.float32)


_epilogue = pl.pallas_call(
    _epilogue_block,
    grid=(L,),
    in_specs=[pl.BlockSpec((1024, 128), lambda i: (i, 0))],
    out_specs=pl.BlockSpec((1, EMBED_DIM, B), lambda i: (i, 0, 0)),
    out_shape=jax.ShapeDtypeStruct((L, EMBED_DIM, B), jnp.float32),
)


def _make_sc_lookup(chunk: int):
    per_w = F // NUM_WORKERS
    assert per_w % chunk == 0 and chunk % 8 == 0
    n_chunks = per_w // chunk
    out_rows = chunk // 4
    mesh = plsc.VectorSubcoreMesh(core_axis_name="c", subcore_axis_name="s")

    @functools.partial(
        pl.kernel,
        mesh=mesh,
        out_type=jax.ShapeDtypeStruct((F // 4, 4 * EMBED_DIM), jnp.float32),
        compiler_params=pltpu.CompilerParams(use_tc_tiling_on_sc=False),
        scratch_types=(
            [pltpu.VMEM((chunk,), jnp.int32)] * 2
            + [pltpu.VMEM((chunk,), jnp.int32)] * 2
            + [pltpu.VMEM((chunk, EMBED_DIM), jnp.float32)] * 2
            + [pltpu.VMEM((chunk, EMBED_DIM), jnp.float32)] * 2
            + [pltpu.VMEM((out_rows, 4 * EMBED_DIM), jnp.float32)] * 2
            + [pltpu.SemaphoreType.DMA] * 3
        ),
    )
    def sc_lookup(ids_hbm, uidx_hbm, g_hbm, u_hbm, out_hbm,
                  idx0, idx1, uidx0, uidx1, g0, g1, u0, u1, o0, o1,
                  sem_i, sem_g, sem_o):
        idx_v = (idx0, idx1)
        uidx_v = (uidx0, uidx1)
        g_v = (g0, g1)
        u_v = (u0, u1)
        out_v = (o0, o1)
        wid = lax.axis_index("s") * NUM_CORES + lax.axis_index("c")
        base = wid * per_w

        def fire_idx(k):
            row0 = pl.multiple_of(base + k * chunk, 8)
            p = k % 2
            return (pltpu.async_copy(ids_hbm.at[pl.ds(row0, chunk)],
                                     idx_v[p], sem_i),
                    pltpu.async_copy(uidx_hbm.at[pl.ds(row0, chunk)],
                                     uidx_v[p], sem_i))

        def fire_gather(k):
            p = k % 2
            return (pltpu.async_copy(g_hbm.at[idx_v[p]], g_v[p], sem_g),
                    pltpu.async_copy(u_hbm.at[uidx_v[p]], u_v[p], sem_g))

        pend_i = fire_idx(0)
        for cp in pend_i:
            cp.wait()
        cur_g = fire_gather(0)
        pend_i = fire_idx(1) if n_chunks > 1 else None
        pend_o = [None, None]
        for k in range(n_chunks):
            p = k % 2
            next_g = None
            if k + 1 < n_chunks:
                for cp in pend_i:
                    cp.wait()
                next_g = fire_gather(k + 1)
            for cp in cur_g:
                cp.wait()
            if k + 2 < n_chunks:
                pend_i = fire_idx(k + 2)
            if pend_o[p] is not None:
                pend_o[p].wait()

            def row_sum(rr, _):
                for a in range(4):
                    for h in (0, LANES):
                        s = pl.ds(h, LANES)
                        d = pl.ds(a * EMBED_DIM + h, LANES)
                        out_v[p][rr, d] = (g_v[p][4 * rr + a, s]
                                           + u_v[p][4 * rr + a, s])
                return 0

            lax.fori_loop(0, out_rows, row_sum, 0, unroll=4)
            orow0 = pl.multiple_of((base + k * chunk) // 4, 8)
            pend_o[p] = pltpu.async_copy(
                out_v[p], out_hbm.at[pl.ds(orow0, out_rows)], sem_o)
            cur_g = next_g
        for po in pend_o:
            if po is not None:
                po.wait()

    return sc_lookup


def kernel(item_ids, user_ids, item_table, channel_tables, user_table):
    V = item_table.shape[0]
    nb_g = pl.cdiv(V, BLK)
    VU_pad = ((user_table.shape[0] + BLK - 1) // BLK) * BLK
    nb_u = VU_pad // BLK

    it_t = item_table.T
    ch_t = jnp.swapaxes(channel_tables, 1, 2)
    us_t = jnp.pad(user_table.T, ((0, 0), (0, VU_pad - user_table.shape[0])))

    it_spec = pl.BlockSpec((EMBED_DIM, BLK), lambda i: (0, i))
    ch_spec = pl.BlockSpec((3, EMBED_DIM, BLK), lambda i: (0, 0, i))
    g_c = _compact_tables(_fuse4_block, [it_spec, ch_spec], nb_g)(it_t, ch_t)
    u_c = _compact_tables(_compact1_block, [it_spec], nb_u)(us_t)

    ids_lv = (item_ids.T.reshape(L, 4, B // 4).transpose(0, 2, 1)
              .reshape(L, B))
    flat_ids = _perm(ids_lv.reshape(F))
    u_v = user_ids.reshape(4, B // 4).T.reshape(B)
    uidx = _perm(jnp.broadcast_to(u_v, (L, B)).reshape(F))

    lookup = _make_sc_lookup(chunk=640)
    packed = lookup(flat_ids, uidx,
                    g_c.reshape(nb_g * BLK, EMBED_DIM),
                    u_c.reshape(VU_pad, EMBED_DIM))
    out_phys = _epilogue(packed)
    return jnp.transpose(out_phys, (2, 0, 1))


# speedup vs baseline: 21.5728x; 1.0451x over previous
import functools

import jax
import jax.numpy as jnp
from jax import lax
from jax.experimental import pallas as pl
from jax.experimental.pallas import tpu as pltpu
from jax.experimental.pallas import tpu_sc as plsc

EMBED_DIM = 32
NUM_CORES = 2
NUM_SUBCORES = 16
NUM_WORKERS = NUM_CORES * NUM_SUBCORES
LANES = 16
BLK = 32768
SUB = 8192
B = 4096
L = 50
F = B * L


def _transpose_mxu(x):
    eye = jnp.eye(EMBED_DIM, dtype=jnp.float32)
    return lax.dot_general(x, eye, (((0,), (0,)), ((), ())),
                           preferred_element_type=jnp.float32)


def _store_transposed(x, out_ref):
    y = _transpose_mxu(x)
    for a in range(4):
        out_ref[:, a * EMBED_DIM:(a + 1) * EMBED_DIM] = (
            y[a * SUB:(a + 1) * SUB, :])


def _fuse4_block(a_ref, ch_ref, out_ref):
    _store_transposed(a_ref[...] + ch_ref[0] + ch_ref[1] + ch_ref[2], out_ref)


def _compact1_block(a_ref, out_ref):
    _store_transposed(a_ref[...], out_ref)


def _compact_tables(tc_body, in_specs, n_blocks):
    out_spec = pl.BlockSpec((SUB, 4 * EMBED_DIM), lambda i: (i, 0))
    return pl.pallas_call(
        tc_body,
        grid=(n_blocks,),
        in_specs=in_specs,
        out_specs=out_spec,
        out_shape=jax.ShapeDtypeStruct((n_blocks * SUB, 4 * EMBED_DIM),
                                       jnp.float32),
    )


def _perm(idx):
    blk = idx // BLK
    rem = idx - blk * BLK
    return blk * BLK + (rem % SUB) * 4 + rem // SUB


EPI_L = 2


def _epilogue_block(x_ref, out_ref):
    eye = jnp.eye(EMBED_DIM, dtype=jnp.float32)
    for j in range(EPI_L):
        for a in range(4):
            xa = x_ref[j * 1024:(j + 1) * 1024,
                       a * EMBED_DIM:(a + 1) * EMBED_DIM]
            out_ref[j, :, a * 1024:(a + 1) * 1024] = lax.dot_general(
                eye, xa, (((1,), (1,)), ((), ())),
                preferred_element_type=jnp.float32)


_epilogue = pl.pallas_call(
    _epilogue_block,
    grid=(L // EPI_L,),
    in_specs=[pl.BlockSpec((EPI_L * 1024, 128), lambda i: (i, 0))],
    out_specs=pl.BlockSpec((EPI_L, EMBED_DIM, B), lambda i: (i, 0, 0)),
    out_shape=jax.ShapeDtypeStruct((L, EMBED_DIM, B), jnp.float32),
)


def _make_sc_lookup(chunk: int):
    per_w = F // NUM_WORKERS
    assert per_w % chunk == 0 and chunk % 8 == 0
    n_chunks = per_w // chunk
    out_rows = chunk // 4
    mesh = plsc.VectorSubcoreMesh(core_axis_name="c", subcore_axis_name="s")

    @functools.partial(
        pl.kernel,
        mesh=mesh,
        out_type=jax.ShapeDtypeStruct((F // 4, 4 * EMBED_DIM), jnp.float32),
        compiler_params=pltpu.CompilerParams(use_tc_tiling_on_sc=False),
        scratch_types=(
            [pltpu.VMEM((chunk,), jnp.int32)] * 2
            + [pltpu.VMEM((chunk,), jnp.int32)] * 2
            + [pltpu.VMEM((chunk, EMBED_DIM), jnp.float32)] * 2
            + [pltpu.VMEM((chunk, EMBED_DIM), jnp.float32)] * 2
            + [pltpu.VMEM((out_rows, 4 * EMBED_DIM), jnp.float32)] * 2
            + [pltpu.SemaphoreType.DMA] * 3
        ),
    )
    def sc_lookup(ids_hbm, uidx_hbm, g_hbm, u_hbm, out_hbm,
                  idx0, idx1, uidx0, uidx1, g0, g1, u0, u1, o0, o1,
                  sem_i, sem_g, sem_o):
        idx_v = (idx0, idx1)
        uidx_v = (uidx0, uidx1)
        g_v = (g0, g1)
        u_v = (u0, u1)
        out_v = (o0, o1)
        wid = lax.axis_index("s") * NUM_CORES + lax.axis_index("c")
        base = wid * per_w

        def fire_idx(k):
            row0 = pl.multiple_of(base + k * chunk, 8)
            p = k % 2
            return (pltpu.async_copy(ids_hbm.at[pl.ds(row0, chunk)],
                                     idx_v[p], sem_i),
                    pltpu.async_copy(uidx_hbm.at[pl.ds(row0, chunk)],
                                     uidx_v[p], sem_i))

        def fire_gather(k):
            p = k % 2
            return (pltpu.async_copy(g_hbm.at[idx_v[p]], g_v[p], sem_g),
                    pltpu.async_copy(u_hbm.at[uidx_v[p]], u_v[p], sem_g))

        pend_i = fire_idx(0)
        for cp in pend_i:
            cp.wait()
        cur_g = fire_gather(0)
        pend_i = fire_idx(1) if n_chunks > 1 else None
        pend_o = [None, None]
        for k in range(n_chunks):
            p = k % 2
            next_g = None
            if k + 1 < n_chunks:
                for cp in pend_i:
                    cp.wait()
                next_g = fire_gather(k + 1)
            for cp in cur_g:
                cp.wait()
            if k + 2 < n_chunks:
                pend_i = fire_idx(k + 2)
            if pend_o[p] is not None:
                pend_o[p].wait()

            def row_sum(rr, _):
                for a in range(4):
                    for h in (0, LANES):
                        s = pl.ds(h, LANES)
                        d = pl.ds(a * EMBED_DIM + h, LANES)
                        out_v[p][rr, d] = (g_v[p][4 * rr + a, s]
                                           + u_v[p][4 * rr + a, s])
                return 0

            lax.fori_loop(0, out_rows, row_sum, 0, unroll=4)
            orow0 = pl.multiple_of((base + k * chunk) // 4, 8)
            pend_o[p] = pltpu.async_copy(
                out_v[p], out_hbm.at[pl.ds(orow0, out_rows)], sem_o)
            cur_g = next_g
        for po in pend_o:
            if po is not None:
                po.wait()

    return sc_lookup


def kernel(item_ids, user_ids, item_table, channel_tables, user_table):
    V = item_table.shape[0]
    nb_g = pl.cdiv(V, BLK)
    VU_pad = ((user_table.shape[0] + BLK - 1) // BLK) * BLK
    nb_u = VU_pad // BLK

    it_t = item_table.T
    ch_t = jnp.swapaxes(channel_tables, 1, 2)
    us_t = jnp.pad(user_table.T, ((0, 0), (0, VU_pad - user_table.shape[0])))

    it_spec = pl.BlockSpec((EMBED_DIM, BLK), lambda i: (0, i))
    ch_spec = pl.BlockSpec((3, EMBED_DIM, BLK), lambda i: (0, 0, i))
    g_c = _compact_tables(_fuse4_block, [it_spec, ch_spec], nb_g)(it_t, ch_t)
    u_c = _compact_tables(_compact1_block, [it_spec], nb_u)(us_t)

    ids_lv = (item_ids.T.reshape(L, 4, B // 4).transpose(0, 2, 1)
              .reshape(L, B))
    flat_ids = _perm(ids_lv.reshape(F))
    u_v = user_ids.reshape(4, B // 4).T.reshape(B)
    uidx = _perm(jnp.broadcast_to(u_v, (L, B)).reshape(F))

    lookup = _make_sc_lookup(chunk=640)
    packed = lookup(flat_ids, uidx,
                    g_c.reshape(nb_g * BLK, EMBED_DIM),
                    u_c.reshape(VU_pad, EMBED_DIM))
    out_phys = _epilogue(packed)
    return jnp.transpose(out_phys, (2, 0, 1))


# speedup vs baseline: 22.1119x; 1.0250x over previous
import functools

import jax
import jax.numpy as jnp
from jax import lax
from jax.experimental import pallas as pl
from jax.experimental.pallas import tpu as pltpu
from jax.experimental.pallas import tpu_sc as plsc

EMBED_DIM = 32
NUM_CORES = 2
NUM_SUBCORES = 16
NUM_WORKERS = NUM_CORES * NUM_SUBCORES
LANES = 16
BLK = 32768
SUB = 8192
B = 4096
L = 50
F = B * L


def _transpose_mxu(x):
    eye = jnp.eye(EMBED_DIM, dtype=jnp.float32)
    return lax.dot_general(x, eye, (((0,), (0,)), ((), ())),
                           preferred_element_type=jnp.float32)


def _store_transposed(x, out_ref):
    y = _transpose_mxu(x)
    for a in range(4):
        out_ref[:, a * EMBED_DIM:(a + 1) * EMBED_DIM] = (
            y[a * SUB:(a + 1) * SUB, :])


def _fuse4_block(a_ref, ch_ref, out_ref):
    _store_transposed(a_ref[...] + ch_ref[0] + ch_ref[1] + ch_ref[2], out_ref)


def _compact1_block(a_ref, out_ref):
    _store_transposed(a_ref[...], out_ref)


def _compact_tables(tc_body, in_specs, n_blocks):
    out_spec = pl.BlockSpec((SUB, 4 * EMBED_DIM), lambda i: (i, 0))
    return pl.pallas_call(
        tc_body,
        grid=(n_blocks,),
        in_specs=in_specs,
        out_specs=out_spec,
        out_shape=jax.ShapeDtypeStruct((n_blocks * SUB, 4 * EMBED_DIM),
                                       jnp.float32),
    )


def _perm(idx):
    blk = idx // BLK
    rem = idx - blk * BLK
    return blk * BLK + (rem % SUB) * 4 + rem // SUB


EPI_L = 2


def _epilogue_block(x_ref, out_ref):
    eye = jnp.eye(EMBED_DIM, dtype=jnp.float32)
    for j in range(EPI_L):
        for a in range(4):
            xa = x_ref[j * 1024:(j + 1) * 1024,
                       a * EMBED_DIM:(a + 1) * EMBED_DIM]
            out_ref[j, :, a * 1024:(a + 1) * 1024] = lax.dot_general(
                eye, xa, (((1,), (1,)), ((), ())),
                preferred_element_type=jnp.float32)


_epilogue = pl.pallas_call(
    _epilogue_block,
    grid=(L // EPI_L,),
    in_specs=[pl.BlockSpec((EPI_L * 1024, 128), lambda i: (i, 0))],
    out_specs=pl.BlockSpec((EPI_L, EMBED_DIM, B), lambda i: (i, 0, 0)),
    out_shape=jax.ShapeDtypeStruct((L, EMBED_DIM, B), jnp.float32),
)


def _make_sc_lookup(chunk: int):
    per_w = F // NUM_WORKERS
    assert per_w % chunk == 0 and chunk % 8 == 0
    n_chunks = per_w // chunk
    out_rows = chunk // 4
    mesh = plsc.VectorSubcoreMesh(core_axis_name="c", subcore_axis_name="s")

    @functools.partial(
        pl.kernel,
        mesh=mesh,
        out_type=jax.ShapeDtypeStruct((F // 4, 4 * EMBED_DIM), jnp.float32),
        compiler_params=pltpu.CompilerParams(use_tc_tiling_on_sc=False),
        scratch_types=(
            [pltpu.VMEM((chunk,), jnp.int32)] * 2
            + [pltpu.VMEM((chunk,), jnp.int32)] * 2
            + [pltpu.VMEM((chunk, EMBED_DIM), jnp.float32)] * 2
            + [pltpu.VMEM((chunk, EMBED_DIM), jnp.float32)] * 2
            + [pltpu.VMEM((out_rows, 4 * EMBED_DIM), jnp.float32)] * 2
            + [pltpu.SemaphoreType.DMA] * 3
        ),
    )
    def sc_lookup(ids_hbm, uidx_hbm, g_hbm, u_hbm, out_hbm,
                  idx0, idx1, uidx0, uidx1, g0, g1, u0, u1, o0, o1,
                  sem_i, sem_g, sem_o):
        idx_v = (idx0, idx1)
        uidx_v = (uidx0, uidx1)
        g_v = (g0, g1)
        u_v = (u0, u1)
        out_v = (o0, o1)
        wid = lax.axis_index("s") * NUM_CORES + lax.axis_index("c")
        base = wid * per_w

        def fire_idx(k):
            row0 = pl.multiple_of(base + k * chunk, 8)
            p = k % 2
            return (pltpu.async_copy(ids_hbm.at[pl.ds(row0, chunk)],
                                     idx_v[p], sem_i),
                    pltpu.async_copy(uidx_hbm.at[pl.ds(row0, chunk)],
                                     uidx_v[p], sem_i))

        def fire_gather(k):
            p = k % 2
            return (pltpu.async_copy(g_hbm.at[idx_v[p]], g_v[p], sem_g),
                    pltpu.async_copy(u_hbm.at[uidx_v[p]], u_v[p], sem_g))

        pend_i = fire_idx(0)
        for cp in pend_i:
            cp.wait()
        cur_g = fire_gather(0)
        pend_i = fire_idx(1) if n_chunks > 1 else None
        pend_o = [None, None]
        for k in range(n_chunks):
            p = k % 2
            next_g = None
            if k + 1 < n_chunks:
                for cp in pend_i:
                    cp.wait()
                next_g = fire_gather(k + 1)
            for cp in cur_g:
                cp.wait()
            if k + 2 < n_chunks:
                pend_i = fire_idx(k + 2)
            if pend_o[p] is not None:
                pend_o[p].wait()

            def row_sum(rr, _):
                for a in range(4):
                    for h in (0, LANES):
                        s = pl.ds(h, LANES)
                        d = pl.ds(a * EMBED_DIM + h, LANES)
                        out_v[p][rr, d] = (g_v[p][4 * rr + a, s]
                                           + u_v[p][4 * rr + a, s])
                return 0

            lax.fori_loop(0, out_rows, row_sum, 0, unroll=4)
            orow0 = pl.multiple_of((base + k * chunk) // 4, 8)
            pend_o[p] = pltpu.async_copy(
                out_v[p], out_hbm.at[pl.ds(orow0, out_rows)], sem_o)
            cur_g = next_g
        for po in pend_o:
            if po is not None:
                po.wait()

    return sc_lookup


def kernel(item_ids, user_ids, item_table, channel_tables, user_table):
    V = item_table.shape[0]
    nb_g = pl.cdiv(V, BLK)
    VU_pad = ((user_table.shape[0] + BLK - 1) // BLK) * BLK
    nb_u = VU_pad // BLK

    it_t = item_table.T
    ch_t = jnp.swapaxes(channel_tables, 1, 2)
    us_t = user_table.T

    it_spec = pl.BlockSpec((EMBED_DIM, BLK), lambda i: (0, i))
    ch_spec = pl.BlockSpec((3, EMBED_DIM, BLK), lambda i: (0, 0, i))
    g_c = _compact_tables(_fuse4_block, [it_spec, ch_spec], nb_g)(it_t, ch_t)
    u_c = _compact_tables(_compact1_block, [it_spec], nb_u)(us_t)

    ids_lv = (item_ids.T.reshape(L, 4, B // 4).transpose(0, 2, 1)
              .reshape(L, B))
    flat_ids = _perm(ids_lv.reshape(F))
    u_v = user_ids.reshape(4, B // 4).T.reshape(B)
    uidx = _perm(jnp.broadcast_to(u_v, (L, B)).reshape(F))

    lookup = _make_sc_lookup(chunk=640)
    packed = lookup(flat_ids, uidx,
                    g_c.reshape(nb_g * BLK, EMBED_DIM),
                    u_c.reshape(VU_pad, EMBED_DIM))
    out_phys = _epilogue(packed)
    return jnp.transpose(out_phys, (2, 0, 1))
